# baseline (speedup 1.0000x reference)
import functools

import jax
import jax.numpy as jnp
from jax import lax
from jax.experimental import pallas as pl
from jax.experimental.pallas import tpu as pltpu
from jax.experimental.pallas import tpu_sc as plsc

N = 10000
E = 160000
D_IN = 256
D_H = 256
N_CLS = 40

NC = 2
NS = 16
NPAD = 10240
EPAD = 163840
RPT = NPAD // NS
CHUNK = 128

MBLK = 512


def _mesh():
    return plsc.VectorSubcoreMesh(core_axis_name="c", subcore_axis_name="s")


NBUF = 4
NPH = 1


def _agg(u_stack, src2, dst3, zeros_dh, dh, deg_extras=None):
    nch = EPAD // NS // CHUNK
    epw = nch * CHUNK
    ncp = nch // NPH
    epp = ncp * CHUNK
    with_deg = deg_extras is not None

    out_types = [jax.ShapeDtypeStruct((NC * NPAD, dh), jnp.bfloat16)]
    scratch = [
        pltpu.VMEM((epp,), jnp.int32),
        pltpu.VMEM((ncp, CHUNK), jnp.int32),
        pltpu.VMEM((NBUF, CHUNK, dh), jnp.bfloat16),
        pltpu.VMEM_SHARED((NPAD, dh), jnp.bfloat16),
        pltpu.SemaphoreType.DMA,
        pltpu.SemaphoreType.DMA,
    ]
    if with_deg:
        out_types.append(jax.ShapeDtypeStruct((NC * NPAD, 16), jnp.float32))
        scratch += [
            pltpu.VMEM((CHUNK, 16), jnp.float32),
            pltpu.VMEM_SHARED((NPAD, 16), jnp.float32),
            pltpu.SemaphoreType.DMA,
        ]

    @functools.partial(
        pl.kernel,
        out_type=out_types if with_deg else out_types[0],
        mesh=_mesh(),
        compiler_params=pltpu.CompilerParams(use_tc_tiling_on_sc=False),
        scratch_types=scratch,
    )
    def agg_kernel(u_hbm, src_hbm, dst_hbm, zeros_hbm, *rest):
        if with_deg:
            (ones_hbm, zeros16_hbm, out_hbm, deg_hbm,
             sidx, didx, rows, acc, gsem, ssem, ones_v, dacc, dsem) = rest
        else:
            out_hbm, sidx, didx, rows, acc, gsem, ssem = rest
        c = lax.axis_index("c")
        s = lax.axis_index("s")
        pltpu.sync_copy(zeros_hbm, acc.at[pl.ds(s * RPT, RPT)])
        if with_deg:
            pltpu.sync_copy(ones_hbm, ones_v)
            pltpu.sync_copy(zeros16_hbm, dacc.at[pl.ds(s * RPT, RPT)])
        plsc.subcore_barrier()

        def gather(k, b):
            pltpu.async_copy(
                u_hbm.at[sidx.at[pl.ds(k * CHUNK, CHUNK)]], rows.at[b], gsem)

        def wait_gather(b):
            pltpu.make_async_copy(
                u_hbm.at[sidx.at[pl.ds(0, CHUNK)]], rows.at[b], gsem).wait()

        def scatter(k, b):
            pltpu.async_copy(rows.at[b], acc.at[didx.at[k]], ssem, add=True)

        def wait_scatter(k, b):
            pltpu.make_async_copy(
                rows.at[b], acc.at[didx.at[k]], ssem).wait()

        def phase(p, carry):
            pltpu.sync_copy(
                src_hbm.at[pl.ds(c * EPAD + s * epw + p * epp, epp)], sidx)
            pltpu.sync_copy(dst_hbm.at[pl.ds(s * nch + p * ncp, ncp)], didx)

            for b in range(NBUF - 1):
                gather(b, b)

            def body(g, carry2):
                for b in range(NBUF):
                    k = g * NBUF + b
                    wait_gather(b)
                    scatter(k, b)
                    if with_deg:
                        pltpu.async_copy(ones_v, dacc.at[didx.at[k]], dsem,
                                         add=True)

                    @pl.when(k >= 1)
                    def _():
                        wait_scatter(k - 1, (b + NBUF - 1) % NBUF)
                        if with_deg:
                            pltpu.make_async_copy(
                                ones_v, dacc.at[didx.at[k - 1]], dsem).wait()

                    @pl.when(k + NBUF - 1 <= ncp - 1)
                    def _():
                        gather(k + NBUF - 1, (b + NBUF - 1) % NBUF)

                return carry2

            lax.fori_loop(0, ncp // NBUF, body, 0)
            wait_scatter(ncp - 1, (NBUF - 1) % NBUF)
            if with_deg:
                pltpu.make_async_copy(
                    ones_v, dacc.at[didx.at[ncp - 1]], dsem).wait()
            return carry

        lax.fori_loop(0, NPH, phase, 0)
        plsc.subcore_barrier()
        pltpu.sync_copy(acc.at[pl.ds(s * RPT, RPT)],
                        out_hbm.at[pl.ds(c * NPAD + s * RPT, RPT)])
        if with_deg:
            pltpu.sync_copy(dacc.at[pl.ds(s * RPT, RPT)],
                            deg_hbm.at[pl.ds(c * NPAD + s * RPT, RPT)])

    if with_deg:
        return agg_kernel(u_stack, src2, dst3, zeros_dh, *deg_extras)
    return agg_kernel(u_stack, src2, dst3, zeros_dh)


def _mm_first(x_p, Ws, Wn):
    def body(x_ref, ws_ref, wn_ref, o_ref):
        w = (ws_ref[...] + wn_ref[...]).astype(jnp.bfloat16)
        u = jnp.dot(x_ref[...].astype(jnp.bfloat16), w,
                    preferred_element_type=jnp.float32)
        ub = u.astype(jnp.bfloat16)
        o_ref[0] = ub[:, :128]
        o_ref[1] = ub[:, 128:]

    return pl.pallas_call(
        body,
        grid=(NPAD // MBLK,),
        in_specs=[
            pl.BlockSpec((MBLK, 256), lambda g: (g, 0)),
            pl.BlockSpec((256, 256), lambda g: (0, 0)),
            pl.BlockSpec((256, 256), lambda g: (0, 0)),
        ],
        out_specs=pl.BlockSpec((2, MBLK, 128), lambda g: (0, g, 0)),
        out_shape=jax.ShapeDtypeStruct((2, NPAD, 128), jnp.bfloat16),
    )(x_p, Ws, Wn)


def _mm_mid(s_stack, deg_stack, Ws, Wn, b2d, dout):
    nb = NPAD // MBLK
    dh2 = dout // 2

    def body(s0_ref, s1_ref, d_ref, ws_ref, wn_ref, b_ref, o_ref):
        deg = jnp.maximum(d_ref[:, 0:1], 1.0)
        bb = b_ref[...]
        z0 = jnp.maximum(s0_ref[...].astype(jnp.float32) / deg + bb[:, :128],
                         0.0).astype(jnp.bfloat16)
        z1 = jnp.maximum(s1_ref[...].astype(jnp.float32) / deg + bb[:, 128:],
                         0.0).astype(jnp.bfloat16)
        w = (ws_ref[...] + wn_ref[...]).astype(jnp.bfloat16)
        u = (jnp.dot(z0, w[:128], preferred_element_type=jnp.float32)
             + jnp.dot(z1, w[128:], preferred_element_type=jnp.float32))
        ub = u.astype(jnp.bfloat16)
        o_ref[0] = ub[:, :dh2]
        o_ref[1] = ub[:, dh2:]

    return pl.pallas_call(
        body,
        grid=(nb,),
        in_specs=[
            pl.BlockSpec((MBLK, 128), lambda g: (g, 0)),
            pl.BlockSpec((MBLK, 128), lambda g: (g + nb, 0)),
            pl.BlockSpec((MBLK, 16), lambda g: (g, 0)),
            pl.BlockSpec((256, dout), lambda g: (0, 0)),
            pl.BlockSpec((256, dout), lambda g: (0, 0)),
            pl.BlockSpec((1, 256), lambda g: (0, 0)),
        ],
        out_specs=pl.BlockSpec((2, MBLK, dh2), lambda g: (0, g, 0)),
        out_shape=jax.ShapeDtypeStruct((2, NPAD, dh2), jnp.bfloat16),
    )(s_stack, s_stack, deg_stack, Ws, Wn, b2d)


def _fin(s_stack, deg_stack, b2d):
    nb = NPAD // MBLK

    def body(s0_ref, s1_ref, d_ref, b_ref, o_ref):
        deg = jnp.maximum(d_ref[:, 0:1], 1.0)
        bb = b_ref[...]
        o_ref[:, :32] = s0_ref[...].astype(jnp.float32) / deg + bb[:, :32]
        o_ref[:, 32:] = s1_ref[...].astype(jnp.float32) / deg + bb[:, 32:]

    return pl.pallas_call(
        body,
        grid=(nb,),
        in_specs=[
            pl.BlockSpec((MBLK, 32), lambda g: (g, 0)),
            pl.BlockSpec((MBLK, 32), lambda g: (g + nb, 0)),
            pl.BlockSpec((MBLK, 16), lambda g: (g, 0)),
            pl.BlockSpec((1, 64), lambda g: (0, 0)),
        ],
        out_specs=pl.BlockSpec((MBLK, 64), lambda g: (g, 0)),
        out_shape=jax.ShapeDtypeStruct((NPAD, 64), jnp.float32),
    )(s_stack, s_stack, deg_stack, b2d)


def kernel(inputs, edge_index, W_self0, W_neigh0, b0,
           W_self1, W_neigh1, b1, W_self2, W_neigh2, b2):
    x = inputs
    src = edge_index[0]
    dst = edge_index[1]
    src_p = jnp.concatenate([src, jnp.zeros((EPAD - E,), jnp.int32)])
    dst_p = jnp.concatenate([dst, jnp.full((EPAD - E,), NPAD - 1, jnp.int32)])
    x_p = jnp.pad(x, ((0, NPAD - N), (0, 0)))

    ones16 = jnp.ones((CHUNK, 16), jnp.float32)
    zeros16 = jnp.zeros((RPT, 16), jnp.float32)
    zeros128 = jnp.zeros((RPT, 128), jnp.bfloat16)
    zeros32 = jnp.zeros((RPT, 32), jnp.bfloat16)

    W2s = jnp.pad(W_self2, ((0, 0), (0, 64 - N_CLS)))
    W2n = jnp.pad(W_neigh2, ((0, 0), (0, 64 - N_CLS)))
    b0_2d = b0.reshape(1, 256)
    b1_2d = b1.reshape(1, 256)
    b2_2d = jnp.pad(b2, (0, 64 - N_CLS)).reshape(1, 64)

    src2 = jnp.concatenate([src_p, src_p + NPAD])
    dst3 = dst_p.reshape(EPAD // CHUNK, CHUNK)

    u0 = _mm_first(x_p, W_self0, W_neigh0).reshape(2 * NPAD, 128)
    s0, deg_stack = _agg(u0, src2, dst3, zeros128, 128,
                         deg_extras=(ones16, zeros16))

    u1 = _mm_mid(s0, deg_stack, W_self1, W_neigh1, b0_2d, 256)
    s1 = _agg(u1.reshape(2 * NPAD, 128), src2, dst3, zeros128, 128)

    u2 = _mm_mid(s1, deg_stack, W2s, W2n, b1_2d, 64)
    s2 = _agg(u2.reshape(2 * NPAD, 32), src2, dst3, zeros32, 32)

    out = _fin(s2, deg_stack, b2_2d)
    return out[:N, :N_CLS]


# speedup vs baseline: 1.0507x; 1.0507x over previous
import functools

import jax
import jax.numpy as jnp
from jax import lax
from jax.experimental import pallas as pl
from jax.experimental.pallas import tpu as pltpu
from jax.experimental.pallas import tpu_sc as plsc

N = 10000
E = 160000
D_IN = 256
D_H = 256
N_CLS = 40

NC = 2
NS = 16
NPAD = 10240
EPAD = 163840
RPT = NPAD // NS
CHUNK = 128

MBLK = 512


def _mesh():
    return plsc.VectorSubcoreMesh(core_axis_name="c", subcore_axis_name="s")


NBUF = 4
NPH = 1


def _agg(u_stack, src2, dst3, zeros_dh, dh):
    nch = EPAD // NS // CHUNK
    epw = nch * CHUNK
    ncp = nch // NPH
    epp = ncp * CHUNK

    @functools.partial(
        pl.kernel,
        out_type=jax.ShapeDtypeStruct((NC * NPAD, dh), jnp.bfloat16),
        mesh=_mesh(),
        compiler_params=pltpu.CompilerParams(use_tc_tiling_on_sc=False),
        scratch_types=[
            pltpu.VMEM((epp,), jnp.int32),
            pltpu.VMEM((ncp, CHUNK), jnp.int32),
            pltpu.VMEM((NBUF, CHUNK, dh), jnp.bfloat16),
            pltpu.VMEM_SHARED((NPAD, dh), jnp.bfloat16),
            pltpu.SemaphoreType.DMA,
            pltpu.SemaphoreType.DMA,
        ],
    )
    def agg_kernel(u_hbm, src_hbm, dst_hbm, zeros_hbm, out_hbm,
                   sidx, didx, rows, acc, gsem, ssem):
        c = lax.axis_index("c")
        s = lax.axis_index("s")
        pltpu.sync_copy(zeros_hbm, acc.at[pl.ds(s * RPT, RPT)])
        plsc.subcore_barrier()

        def gather(k, b):
            pltpu.async_copy(
                u_hbm.at[sidx.at[pl.ds(k * CHUNK, CHUNK)]], rows.at[b], gsem)

        def wait_gather(b):
            pltpu.make_async_copy(
                u_hbm.at[sidx.at[pl.ds(0, CHUNK)]], rows.at[b], gsem).wait()

        def scatter(k, b):
            pltpu.async_copy(rows.at[b], acc.at[didx.at[k]], ssem, add=True)

        def wait_scatter(k, b):
            pltpu.make_async_copy(
                rows.at[b], acc.at[didx.at[k]], ssem).wait()

        def phase(p, carry):
            pltpu.sync_copy(
                src_hbm.at[pl.ds(c * EPAD + s * epw + p * epp, epp)], sidx)
            pltpu.sync_copy(dst_hbm.at[pl.ds(s * nch + p * ncp, ncp)], didx)

            for b in range(NBUF - 1):
                gather(b, b)

            def body(g, carry2):
                for b in range(NBUF):
                    k = g * NBUF + b
                    wait_gather(b)
                    scatter(k, b)

                    @pl.when(k >= 1)
                    def _():
                        wait_scatter(k - 1, (b + NBUF - 1) % NBUF)

                    @pl.when(k + NBUF - 1 <= ncp - 1)
                    def _():
                        gather(k + NBUF - 1, (b + NBUF - 1) % NBUF)

                return carry2

            lax.fori_loop(0, ncp // NBUF, body, 0)
            wait_scatter(ncp - 1, (NBUF - 1) % NBUF)
            return carry

        lax.fori_loop(0, NPH, phase, 0)
        plsc.subcore_barrier()
        pltpu.sync_copy(acc.at[pl.ds(s * RPT, RPT)],
                        out_hbm.at[pl.ds(c * NPAD + s * RPT, RPT)])

    return agg_kernel(u_stack, src2, dst3, zeros_dh)


def _deg(dst_p, ones16, zeros16):
    nch = EPAD // (NC * NS) // CHUNK

    @functools.partial(
        pl.kernel,
        out_type=jax.ShapeDtypeStruct((NC * NPAD, 16), jnp.float32),
        mesh=_mesh(),
        compiler_params=pltpu.CompilerParams(use_tc_tiling_on_sc=False),
        scratch_types=[
            pltpu.VMEM((CHUNK,), jnp.int32),
            pltpu.VMEM((CHUNK, 16), jnp.float32),
            pltpu.VMEM_SHARED((NPAD, 16), jnp.float32),
        ],
    )
    def deg_kernel(dst_hbm, ones_hbm, zeros_hbm, out_hbm, didx, ones_v, acc):
        c = lax.axis_index("c")
        s = lax.axis_index("s")
        pltpu.sync_copy(ones_hbm, ones_v)
        pltpu.sync_copy(zeros_hbm, acc.at[pl.ds(s * RPT, RPT)])
        plsc.subcore_barrier()
        base = (c * NS + s) * (nch * CHUNK)

        def body(k, carry):
            pltpu.sync_copy(dst_hbm.at[pl.ds(base + k * CHUNK, CHUNK)], didx)
            pltpu.sync_copy(ones_v, acc.at[didx], add=True)
            return carry

        lax.fori_loop(0, nch, body, 0)
        plsc.subcore_barrier()
        pltpu.sync_copy(acc.at[pl.ds(s * RPT, RPT)],
                        out_hbm.at[pl.ds(c * NPAD + s * RPT, RPT)])

    return deg_kernel(dst_p, ones16, zeros16)


def _mm_first(x_p, Ws, Wn):
    def body(x_ref, ws_ref, wn_ref, o_ref):
        w = (ws_ref[...] + wn_ref[...]).astype(jnp.bfloat16)
        u = jnp.dot(x_ref[...].astype(jnp.bfloat16), w,
                    preferred_element_type=jnp.float32)
        ub = u.astype(jnp.bfloat16)
        o_ref[0] = ub[:, :128]
        o_ref[1] = ub[:, 128:]

    return pl.pallas_call(
        body,
        grid=(NPAD // MBLK,),
        in_specs=[
            pl.BlockSpec((MBLK, 256), lambda g: (g, 0)),
            pl.BlockSpec((256, 256), lambda g: (0, 0)),
            pl.BlockSpec((256, 256), lambda g: (0, 0)),
        ],
        out_specs=pl.BlockSpec((2, MBLK, 128), lambda g: (0, g, 0)),
        out_shape=jax.ShapeDtypeStruct((2, NPAD, 128), jnp.bfloat16),
    )(x_p, Ws, Wn)


def _mm_mid(s_stack, deg_stack, Ws, Wn, b2d, dout):
    nb = NPAD // MBLK
    dh2 = dout // 2

    def body(s0_ref, s1_ref, d0_ref, d1_ref, ws_ref, wn_ref, b_ref, o_ref):
        deg = jnp.maximum(d0_ref[:, 0:1] + d1_ref[:, 0:1], 1.0)
        bb = b_ref[...]
        z0 = jnp.maximum(s0_ref[...].astype(jnp.float32) / deg + bb[:, :128],
                         0.0).astype(jnp.bfloat16)
        z1 = jnp.maximum(s1_ref[...].astype(jnp.float32) / deg + bb[:, 128:],
                         0.0).astype(jnp.bfloat16)
        w = (ws_ref[...] + wn_ref[...]).astype(jnp.bfloat16)
        u = (jnp.dot(z0, w[:128], preferred_element_type=jnp.float32)
             + jnp.dot(z1, w[128:], preferred_element_type=jnp.float32))
        ub = u.astype(jnp.bfloat16)
        o_ref[0] = ub[:, :dh2]
        o_ref[1] = ub[:, dh2:]

    return pl.pallas_call(
        body,
        grid=(nb,),
        in_specs=[
            pl.BlockSpec((MBLK, 128), lambda g: (g, 0)),
            pl.BlockSpec((MBLK, 128), lambda g: (g + nb, 0)),
            pl.BlockSpec((MBLK, 16), lambda g: (g, 0)),
            pl.BlockSpec((MBLK, 16), lambda g: (g + nb, 0)),
            pl.BlockSpec((256, dout), lambda g: (0, 0)),
            pl.BlockSpec((256, dout), lambda g: (0, 0)),
            pl.BlockSpec((1, 256), lambda g: (0, 0)),
        ],
        out_specs=pl.BlockSpec((2, MBLK, dh2), lambda g: (0, g, 0)),
        out_shape=jax.ShapeDtypeStruct((2, NPAD, dh2), jnp.bfloat16),
    )(s_stack, s_stack, deg_stack, deg_stack, Ws, Wn, b2d)


def _fin(s_stack, deg_stack, b2d):
    nb = NPAD // MBLK

    def body(s0_ref, s1_ref, d0_ref, d1_ref, b_ref, o_ref):
        deg = jnp.maximum(d0_ref[:, 0:1] + d1_ref[:, 0:1], 1.0)
        bb = b_ref[...]
        o_ref[:, :32] = s0_ref[...].astype(jnp.float32) / deg + bb[:, :32]
        o_ref[:, 32:] = s1_ref[...].astype(jnp.float32) / deg + bb[:, 32:]

    return pl.pallas_call(
        body,
        grid=(nb,),
        in_specs=[
            pl.BlockSpec((MBLK, 32), lambda g: (g, 0)),
            pl.BlockSpec((MBLK, 32), lambda g: (g + nb, 0)),
            pl.BlockSpec((MBLK, 16), lambda g: (g, 0)),
            pl.BlockSpec((MBLK, 16), lambda g: (g + nb, 0)),
            pl.BlockSpec((1, 64), lambda g: (0, 0)),
        ],
        out_specs=pl.BlockSpec((MBLK, 64), lambda g: (g, 0)),
        out_shape=jax.ShapeDtypeStruct((NPAD, 64), jnp.float32),
    )(s_stack, s_stack, deg_stack, deg_stack, b2d)


def kernel(inputs, edge_index, W_self0, W_neigh0, b0,
           W_self1, W_neigh1, b1, W_self2, W_neigh2, b2):
    x = inputs
    src = edge_index[0]
    dst = edge_index[1]
    src_p = jnp.concatenate([src, jnp.zeros((EPAD - E,), jnp.int32)])
    dst_p = jnp.concatenate([dst, jnp.full((EPAD - E,), NPAD - 1, jnp.int32)])
    x_p = jnp.pad(x, ((0, NPAD - N), (0, 0)))

    ones16 = jnp.ones((CHUNK, 16), jnp.float32)
    zeros16 = jnp.zeros((RPT, 16), jnp.float32)
    zeros128 = jnp.zeros((RPT, 128), jnp.bfloat16)
    zeros32 = jnp.zeros((RPT, 32), jnp.bfloat16)

    W2s = jnp.pad(W_self2, ((0, 0), (0, 64 - N_CLS)))
    W2n = jnp.pad(W_neigh2, ((0, 0), (0, 64 - N_CLS)))
    b0_2d = b0.reshape(1, 256)
    b1_2d = b1.reshape(1, 256)
    b2_2d = jnp.pad(b2, (0, 64 - N_CLS)).reshape(1, 64)

    src2 = jnp.concatenate([src_p, src_p + NPAD])
    dst3 = dst_p.reshape(EPAD // CHUNK, CHUNK)

    deg_stack = _deg(dst_p, ones16, zeros16)
    u0 = _mm_first(x_p, W_self0, W_neigh0).reshape(2 * NPAD, 128)
    s0 = _agg(u0, src2, dst3, zeros128, 128)

    u1 = _mm_mid(s0, deg_stack, W_self1, W_neigh1, b0_2d, 256)
    s1 = _agg(u1.reshape(2 * NPAD, 128), src2, dst3, zeros128, 128)

    u2 = _mm_mid(s1, deg_stack, W2s, W2n, b1_2d, 64)
    s2 = _agg(u2.reshape(2 * NPAD, 32), src2, dst3, zeros32, 32)

    out = _fin(s2, deg_stack, b2_2d)
    return out[:N, :N_CLS]


# speedup vs baseline: 1.0665x; 1.0150x over previous
import functools

import jax
import jax.numpy as jnp
from jax import lax
from jax.experimental import pallas as pl
from jax.experimental.pallas import tpu as pltpu
from jax.experimental.pallas import tpu_sc as plsc

N = 10000
E = 160000
D_IN = 256
D_H = 256
N_CLS = 40

NC = 2
NS = 16
NPAD = 10240
EPAD = 163840
RPT = NPAD // NS
CHUNK = 128

MBLK = 512


def _mesh():
    return plsc.VectorSubcoreMesh(core_axis_name="c", subcore_axis_name="s")


NBUF = 8
NPH = 1


def _agg(u_stack, src2, dst3, zeros_dh, dh):
    nch = EPAD // NS // CHUNK
    epw = nch * CHUNK
    ncp = nch // NPH
    epp = ncp * CHUNK

    @functools.partial(
        pl.kernel,
        out_type=jax.ShapeDtypeStruct((NC * NPAD, dh), jnp.bfloat16),
        mesh=_mesh(),
        compiler_params=pltpu.CompilerParams(use_tc_tiling_on_sc=False),
        scratch_types=[
            pltpu.VMEM((epp,), jnp.int32),
            pltpu.VMEM((ncp, CHUNK), jnp.int32),
            pltpu.VMEM((NBUF, CHUNK, dh), jnp.bfloat16),
            pltpu.VMEM_SHARED((NPAD, dh), jnp.bfloat16),
            pltpu.SemaphoreType.DMA,
            pltpu.SemaphoreType.DMA,
        ],
    )
    def agg_kernel(u_hbm, src_hbm, dst_hbm, zeros_hbm, out_hbm,
                   sidx, didx, rows, acc, gsem, ssem):
        c = lax.axis_index("c")
        s = lax.axis_index("s")
        pltpu.sync_copy(zeros_hbm, acc.at[pl.ds(s * RPT, RPT)])
        plsc.subcore_barrier()

        def gather(k, b):
            pltpu.async_copy(
                u_hbm.at[sidx.at[pl.ds(k * CHUNK, CHUNK)]], rows.at[b], gsem)

        def wait_gather(b):
            pltpu.make_async_copy(
                u_hbm.at[sidx.at[pl.ds(0, CHUNK)]], rows.at[b], gsem).wait()

        def scatter(k, b):
            pltpu.async_copy(rows.at[b], acc.at[didx.at[k]], ssem, add=True)

        def wait_scatter(k, b):
            pltpu.make_async_copy(
                rows.at[b], acc.at[didx.at[k]], ssem).wait()

        def phase(p, carry):
            pltpu.sync_copy(
                src_hbm.at[pl.ds(c * EPAD + s * epw + p * epp, epp)], sidx)
            pltpu.sync_copy(dst_hbm.at[pl.ds(s * nch + p * ncp, ncp)], didx)

            for b in range(NBUF - 1):
                gather(b, b)

            def body(g, carry2):
                for b in range(NBUF):
                    k = g * NBUF + b
                    wait_gather(b)
                    scatter(k, b)

                    @pl.when(k >= 1)
                    def _():
                        wait_scatter(k - 1, (b + NBUF - 1) % NBUF)

                    @pl.when(k + NBUF - 1 <= ncp - 1)
                    def _():
                        gather(k + NBUF - 1, (b + NBUF - 1) % NBUF)

                return carry2

            lax.fori_loop(0, ncp // NBUF, body, 0)
            wait_scatter(ncp - 1, (NBUF - 1) % NBUF)
            return carry

        lax.fori_loop(0, NPH, phase, 0)
        plsc.subcore_barrier()
        pltpu.sync_copy(acc.at[pl.ds(s * RPT, RPT)],
                        out_hbm.at[pl.ds(c * NPAD + s * RPT, RPT)])

    return agg_kernel(u_stack, src2, dst3, zeros_dh)


def _deg(dst_p, ones16, zeros16):
    nch = EPAD // (NC * NS) // CHUNK

    @functools.partial(
        pl.kernel,
        out_type=jax.ShapeDtypeStruct((NC * NPAD, 16), jnp.float32),
        mesh=_mesh(),
        compiler_params=pltpu.CompilerParams(use_tc_tiling_on_sc=False),
        scratch_types=[
            pltpu.VMEM((CHUNK,), jnp.int32),
            pltpu.VMEM((CHUNK, 16), jnp.float32),
            pltpu.VMEM_SHARED((NPAD, 16), jnp.float32),
        ],
    )
    def deg_kernel(dst_hbm, ones_hbm, zeros_hbm, out_hbm, didx, ones_v, acc):
        c = lax.axis_index("c")
        s = lax.axis_index("s")
        pltpu.sync_copy(ones_hbm, ones_v)
        pltpu.sync_copy(zeros_hbm, acc.at[pl.ds(s * RPT, RPT)])
        plsc.subcore_barrier()
        base = (c * NS + s) * (nch * CHUNK)

        def body(k, carry):
            pltpu.sync_copy(dst_hbm.at[pl.ds(base + k * CHUNK, CHUNK)], didx)
            pltpu.sync_copy(ones_v, acc.at[didx], add=True)
            return carry

        lax.fori_loop(0, nch, body, 0)
        plsc.subcore_barrier()
        pltpu.sync_copy(acc.at[pl.ds(s * RPT, RPT)],
                        out_hbm.at[pl.ds(c * NPAD + s * RPT, RPT)])

    return deg_kernel(dst_p, ones16, zeros16)


def _mm_first(x_p, Ws, Wn):
    def body(x_ref, ws_ref, wn_ref, o_ref):
        w = (ws_ref[...] + wn_ref[...]).astype(jnp.bfloat16)
        u = jnp.dot(x_ref[...].astype(jnp.bfloat16), w,
                    preferred_element_type=jnp.float32)
        ub = u.astype(jnp.bfloat16)
        o_ref[0] = ub[:, :128]
        o_ref[1] = ub[:, 128:]

    return pl.pallas_call(
        body,
        grid=(NPAD // MBLK,),
        in_specs=[
            pl.BlockSpec((MBLK, 256), lambda g: (g, 0)),
            pl.BlockSpec((256, 256), lambda g: (0, 0)),
            pl.BlockSpec((256, 256), lambda g: (0, 0)),
        ],
        out_specs=pl.BlockSpec((2, MBLK, 128), lambda g: (0, g, 0)),
        out_shape=jax.ShapeDtypeStruct((2, NPAD, 128), jnp.bfloat16),
    )(x_p, Ws, Wn)


def _mm_mid(s_stack, deg_stack, Ws, Wn, b2d, dout):
    nb = NPAD // MBLK
    dh2 = dout // 2

    def body(s0_ref, s1_ref, d0_ref, d1_ref, ws_ref, wn_ref, b_ref, o_ref):
        deg = jnp.maximum(d0_ref[:, 0:1] + d1_ref[:, 0:1], 1.0)
        bb = b_ref[...]
        z0 = jnp.maximum(s0_ref[...].astype(jnp.float32) / deg + bb[:, :128],
                         0.0).astype(jnp.bfloat16)
        z1 = jnp.maximum(s1_ref[...].astype(jnp.float32) / deg + bb[:, 128:],
                         0.0).astype(jnp.bfloat16)
        w = (ws_ref[...] + wn_ref[...]).astype(jnp.bfloat16)
        u = (jnp.dot(z0, w[:128], preferred_element_type=jnp.float32)
             + jnp.dot(z1, w[128:], preferred_element_type=jnp.float32))
        ub = u.astype(jnp.bfloat16)
        o_ref[0] = ub[:, :dh2]
        o_ref[1] = ub[:, dh2:]

    return pl.pallas_call(
        body,
        grid=(nb,),
        in_specs=[
            pl.BlockSpec((MBLK, 128), lambda g: (g, 0)),
            pl.BlockSpec((MBLK, 128), lambda g: (g + nb, 0)),
            pl.BlockSpec((MBLK, 16), lambda g: (g, 0)),
            pl.BlockSpec((MBLK, 16), lambda g: (g + nb, 0)),
            pl.BlockSpec((256, dout), lambda g: (0, 0)),
            pl.BlockSpec((256, dout), lambda g: (0, 0)),
            pl.BlockSpec((1, 256), lambda g: (0, 0)),
        ],
        out_specs=pl.BlockSpec((2, MBLK, dh2), lambda g: (0, g, 0)),
        out_shape=jax.ShapeDtypeStruct((2, NPAD, dh2), jnp.bfloat16),
    )(s_stack, s_stack, deg_stack, deg_stack, Ws, Wn, b2d)


def _fin(s_stack, deg_stack, b2d):
    nb = NPAD // MBLK

    def body(s0_ref, s1_ref, d0_ref, d1_ref, b_ref, o_ref):
        deg = jnp.maximum(d0_ref[:, 0:1] + d1_ref[:, 0:1], 1.0)
        bb = b_ref[...]
        o_ref[:, :32] = s0_ref[...].astype(jnp.float32) / deg + bb[:, :32]
        o_ref[:, 32:] = s1_ref[...].astype(jnp.float32) / deg + bb[:, 32:]

    return pl.pallas_call(
        body,
        grid=(nb,),
        in_specs=[
            pl.BlockSpec((MBLK, 32), lambda g: (g, 0)),
            pl.BlockSpec((MBLK, 32), lambda g: (g + nb, 0)),
            pl.BlockSpec((MBLK, 16), lambda g: (g, 0)),
            pl.BlockSpec((MBLK, 16), lambda g: (g + nb, 0)),
            pl.BlockSpec((1, 64), lambda g: (0, 0)),
        ],
        out_specs=pl.BlockSpec((MBLK, 64), lambda g: (g, 0)),
        out_shape=jax.ShapeDtypeStruct((NPAD, 64), jnp.float32),
    )(s_stack, s_stack, deg_stack, deg_stack, b2d)


def kernel(inputs, edge_index, W_self0, W_neigh0, b0,
           W_self1, W_neigh1, b1, W_self2, W_neigh2, b2):
    x = inputs
    src = edge_index[0]
    dst = edge_index[1]
    src_p = jnp.concatenate([src, jnp.zeros((EPAD - E,), jnp.int32)])
    dst_p = jnp.concatenate([dst, jnp.full((EPAD - E,), NPAD - 1, jnp.int32)])
    x_p = jnp.pad(x, ((0, NPAD - N), (0, 0)))

    ones16 = jnp.ones((CHUNK, 16), jnp.float32)
    zeros16 = jnp.zeros((RPT, 16), jnp.float32)
    zeros128 = jnp.zeros((RPT, 128), jnp.bfloat16)
    zeros32 = jnp.zeros((RPT, 32), jnp.bfloat16)

    W2s = jnp.pad(W_self2, ((0, 0), (0, 64 - N_CLS)))
    W2n = jnp.pad(W_neigh2, ((0, 0), (0, 64 - N_CLS)))
    b0_2d = b0.reshape(1, 256)
    b1_2d = b1.reshape(1, 256)
    b2_2d = jnp.pad(b2, (0, 64 - N_CLS)).reshape(1, 64)

    src2 = jnp.concatenate([src_p, src_p + NPAD])
    dst3 = dst_p.reshape(EPAD // CHUNK, CHUNK)

    deg_stack = _deg(dst_p, ones16, zeros16)
    u0 = _mm_first(x_p, W_self0, W_neigh0).reshape(2 * NPAD, 128)
    s0 = _agg(u0, src2, dst3, zeros128, 128)

    u1 = _mm_mid(s0, deg_stack, W_self1, W_neigh1, b0_2d, 256)
    s1 = _agg(u1.reshape(2 * NPAD, 128), src2, dst3, zeros128, 128)

    u2 = _mm_mid(s1, deg_stack, W2s, W2n, b1_2d, 64)
    s2 = _agg(u2.reshape(2 * NPAD, 32), src2, dst3, zeros32, 32)

    out = _fin(s2, deg_stack, b2_2d)
    return out[:N, :N_CLS]
